# Initial kernel scaffold; baseline (speedup 1.0000x reference)
import jax
import jax.numpy as jnp
from jax.experimental import pallas as pl


def kernel(logits, temperature, top_k, top_p):
    raise NotImplementedError("write your pallas kernel here")


# speedup vs baseline: 35.2367x; 35.2367x over previous
import jax
import jax.numpy as jnp
from jax import lax
from jax.experimental import pallas as pl
from jax.experimental.pallas import tpu as pltpu

_ROWS = 8
_NEG_INF = float("-inf")
_I32_MAX = jnp.iinfo(jnp.int32).max
_TIE_UNROLL = 16


def _floor_avg(a, b):
    return (a & b) + ((a ^ b) >> 1)


def _ceil_avg(a, b):
    return (a & b) + ((a ^ b) >> 1) + ((a ^ b) & 1)


def _sampler_kernel(x_ref, t_ref, k_ref, p_ref, id_ref, o_ref, key_ref, prob_ref, cut_ref):
    G, V = x_ref.shape
    y = x_ref[...] / t_ref[...]
    o_ref[...] = y

    b = lax.bitcast_convert_type(y, jnp.int32)
    key = b ^ (lax.shift_right_arithmetic(b, 31) & jnp.int32(0x7FFFFFFF))
    key_ref[...] = key

    mx = jnp.max(y, axis=1, keepdims=True)
    mx_key = jnp.max(key, axis=1, keepdims=True)
    mn_key = jnp.min(key, axis=1, keepdims=True)

    idx = lax.broadcasted_iota(jnp.int32, (G, V), 1)
    id_ref[...] = jnp.min(jnp.where(y == mx, idx, V), axis=1, keepdims=True)

    k = jnp.clip(k_ref[...], 1, V)
    tp = p_ref[...]

    def body1(_, lohi):
        lo, hi = lohi
        mid = _ceil_avg(lo, hi)
        cnt = jnp.sum((key_ref[...] >= mid).astype(jnp.int32), axis=1, keepdims=True)
        pred = cnt >= k
        return jnp.where(pred, mid, lo), jnp.where(pred, hi, mid - 1)

    kv, _ = lax.fori_loop(0, 32, body1, (mn_key, mx_key))

    e = jnp.where(key_ref[...] >= kv, jnp.exp(y - mx), 0.0)
    z = jnp.sum(e, axis=1, keepdims=True)
    prob_ref[...] = e / z

    def body2(_, lohi):
        lo, hi = lohi
        mid = _floor_avg(lo, hi)
        g = jnp.sum(jnp.where(key_ref[...] > mid, prob_ref[...], 0.0), axis=1, keepdims=True)
        pred = g <= tp
        return jnp.where(pred, lo, mid + 1), jnp.where(pred, mid, hi)

    ws, _ = lax.fori_loop(0, 32, body2, (mn_key, mx_key))

    thr = jnp.maximum(kv, ws)
    keep = key_ref[...] >= thr

    w_cls = jnp.min(jnp.where(keep, key_ref[...], _I32_MAX), axis=1, keepdims=True)
    is_w = key_ref[...] == w_cls
    t_w = jnp.sum(is_w.astype(jnp.int32), axis=1, keepdims=True)
    g_w = jnp.sum(jnp.where(key_ref[...] > w_cls, prob_ref[...], 0.0), axis=1, keepdims=True)
    p_w = jnp.max(jnp.where(is_w, prob_ref[...], 0.0), axis=1, keepdims=True)

    c = jnp.ones_like(t_w)
    s = g_w
    for q in range(2, _TIE_UNROLL + 1):
        s = s + p_w
        c = c + ((s <= tp) & (q <= t_w)).astype(jnp.int32)

    cut_ref[...] = jnp.full((G, 1), V, jnp.int32)

    @pl.when(jnp.any((t_w > 1) & (c < t_w)))
    def _tie_cut():
        last = jnp.full((G, 1), -1, jnp.int32)
        cut = jnp.full((G, 1), V, jnp.int32)
        for q in range(1, _TIE_UNROLL + 1):
            nxt = jnp.min(jnp.where(is_w & (idx > last), idx, V), axis=1, keepdims=True)
            cut = jnp.where(c == q, nxt, cut)
            last = nxt
        cut_ref[...] = cut

    keep = keep & ~(is_w & (idx > cut_ref[...]))
    o_ref[...] = jnp.where(keep, o_ref[...], _NEG_INF)


def kernel(logits, temperature, top_k, top_p):
    B, V = logits.shape
    logits = logits.astype(jnp.float32)
    grid = (B // _ROWS,)
    row_spec = pl.BlockSpec((_ROWS, 1), lambda i: (i, 0))
    ids, out = pl.pallas_call(
        _sampler_kernel,
        grid=grid,
        in_specs=[
            pl.BlockSpec((_ROWS, V), lambda i: (i, 0)),
            row_spec,
            row_spec,
            row_spec,
        ],
        out_specs=[row_spec, pl.BlockSpec((_ROWS, V), lambda i: (i, 0))],
        out_shape=[
            jax.ShapeDtypeStruct((B, 1), jnp.int32),
            jax.ShapeDtypeStruct((B, V), jnp.float32),
        ],
        scratch_shapes=[
            pltpu.VMEM((_ROWS, V), jnp.int32),
            pltpu.VMEM((_ROWS, V), jnp.float32),
            pltpu.VMEM((_ROWS, 1), jnp.int32),
        ],
    )(
        logits,
        temperature.astype(jnp.float32)[:, None],
        top_k.astype(jnp.int32)[:, None],
        top_p.astype(jnp.float32)[:, None],
    )
    return ids, out


# speedup vs baseline: 36.5385x; 1.0369x over previous
import jax
import jax.numpy as jnp
from jax import lax
from jax.experimental import pallas as pl
from jax.experimental.pallas import tpu as pltpu

_ROWS = 8
_NEG_INF = float("-inf")
_I32_MAX = jnp.iinfo(jnp.int32).max
_TIE_UNROLL = 16


def _floor_avg(a, b):
    return (a & b) + ((a ^ b) >> 1)


def _ceil_avg(a, b):
    return (a & b) + ((a ^ b) >> 1) + ((a ^ b) & 1)


def _sampler_kernel(x_ref, t_ref, k_ref, p_ref, id_ref, o_ref, key_ref, prob_ref, cut_ref):
    G, V = x_ref.shape
    y = x_ref[...] / t_ref[...]
    o_ref[...] = y

    b = lax.bitcast_convert_type(y, jnp.int32)
    key = b ^ (lax.shift_right_arithmetic(b, 31) & jnp.int32(0x7FFFFFFF))
    key_ref[...] = key

    mx = jnp.max(y, axis=1, keepdims=True)
    mx_key = jnp.max(key, axis=1, keepdims=True)
    mn_key = jnp.min(key, axis=1, keepdims=True)

    idx = lax.broadcasted_iota(jnp.int32, (G, V), 1)
    id_ref[...] = jnp.min(jnp.where(y == mx, idx, V), axis=1, keepdims=True)

    k = jnp.clip(k_ref[...], 1, V)
    tp = p_ref[...]

    def body1(lohi):
        lo, hi = lohi
        mid = _ceil_avg(lo, hi)
        cnt = jnp.sum((key_ref[...] >= mid).astype(jnp.int32), axis=1, keepdims=True)
        pred = cnt >= k
        return jnp.where(pred, mid, lo), jnp.where(pred, hi, mid - 1)

    def not_done(lohi):
        lo, hi = lohi
        return jnp.any(lo < hi)

    kv, _ = lax.while_loop(not_done, body1, (mn_key, mx_key))

    e = jnp.where(key_ref[...] >= kv, jnp.exp(y - mx), 0.0)
    z = jnp.sum(e, axis=1, keepdims=True)
    prob_ref[...] = e / z

    def body2(lohi):
        lo, hi = lohi
        mid = _floor_avg(lo, hi)
        g = jnp.sum(jnp.where(key_ref[...] > mid, prob_ref[...], 0.0), axis=1, keepdims=True)
        pred = g <= tp
        return jnp.where(pred, lo, mid + 1), jnp.where(pred, mid, hi)

    ws, _ = lax.while_loop(not_done, body2, (kv, mx_key))

    thr = jnp.maximum(kv, ws)
    keep = key_ref[...] >= thr

    w_cls = jnp.min(jnp.where(keep, key_ref[...], _I32_MAX), axis=1, keepdims=True)
    is_w = key_ref[...] == w_cls
    t_w = jnp.sum(is_w.astype(jnp.int32), axis=1, keepdims=True)
    g_w = jnp.sum(jnp.where(key_ref[...] > w_cls, prob_ref[...], 0.0), axis=1, keepdims=True)
    p_w = jnp.max(jnp.where(is_w, prob_ref[...], 0.0), axis=1, keepdims=True)

    c = jnp.ones_like(t_w)
    s = g_w
    for q in range(2, _TIE_UNROLL + 1):
        s = s + p_w
        c = c + ((s <= tp) & (q <= t_w)).astype(jnp.int32)

    cut_ref[...] = jnp.full((G, 1), V, jnp.int32)

    @pl.when(jnp.any((t_w > 1) & (c < t_w)))
    def _tie_cut():
        last = jnp.full((G, 1), -1, jnp.int32)
        cut = jnp.full((G, 1), V, jnp.int32)
        for q in range(1, _TIE_UNROLL + 1):
            nxt = jnp.min(jnp.where(is_w & (idx > last), idx, V), axis=1, keepdims=True)
            cut = jnp.where(c == q, nxt, cut)
            last = nxt
        cut_ref[...] = cut

    keep = keep & ~(is_w & (idx > cut_ref[...]))
    o_ref[...] = jnp.where(keep, o_ref[...], _NEG_INF)


def kernel(logits, temperature, top_k, top_p):
    B, V = logits.shape
    logits = logits.astype(jnp.float32)
    grid = (B // _ROWS,)
    row_spec = pl.BlockSpec((_ROWS, 1), lambda i: (i, 0))
    ids, out = pl.pallas_call(
        _sampler_kernel,
        grid=grid,
        in_specs=[
            pl.BlockSpec((_ROWS, V), lambda i: (i, 0)),
            row_spec,
            row_spec,
            row_spec,
        ],
        out_specs=[row_spec, pl.BlockSpec((_ROWS, V), lambda i: (i, 0))],
        out_shape=[
            jax.ShapeDtypeStruct((B, 1), jnp.int32),
            jax.ShapeDtypeStruct((B, V), jnp.float32),
        ],
        scratch_shapes=[
            pltpu.VMEM((_ROWS, V), jnp.int32),
            pltpu.VMEM((_ROWS, V), jnp.float32),
            pltpu.VMEM((_ROWS, 1), jnp.int32),
        ],
        compiler_params=pltpu.CompilerParams(
            dimension_semantics=("parallel",),
        ),
    )(
        logits,
        temperature.astype(jnp.float32)[:, None],
        top_k.astype(jnp.int32)[:, None],
        top_p.astype(jnp.float32)[:, None],
    )
    return ids, out
